# baseline (device time: 27204 ns/iter reference)
import jax
import jax.numpy as jnp
from jax import lax
from jax.experimental import pallas as pl
from jax.experimental.pallas import tpu as pltpu

N_DEV = 4
NSUB = 4


def kernel(A, B):
    m, k = A.shape
    k2, n = B.shape
    half = m // 2
    quart = m // 4
    nh = n // 2
    nq = nh // NSUB

    def body(a_ref, b_ref, out_ref, r1, r2, send_sems, recv_sems):
        my = lax.axis_index("i")
        x = my // 2
        y = (my % 2) ^ x
        py = my ^ 1
        px = 3 - my

        streams = [
            dict(c0=0, kb=y, qb=x, p1=py, p2=px),
            dict(c0=nh, kb=x, qb=y, p1=px, p2=py),
        ]

        barrier_sem = pltpu.get_barrier_semaphore()
        for nbr in (py, px):
            pl.semaphore_signal(
                barrier_sem, inc=1,
                device_id=(nbr,), device_id_type=pl.DeviceIdType.MESH,
            )

        def block_dot(row_start, c0):
            return jnp.dot(
                a_ref[pl.ds(row_start, half), :].astype(jnp.bfloat16),
                b_ref[:, c0:c0 + nh].astype(jnp.bfloat16),
                preferred_element_type=jnp.float32,
            ).astype(jnp.bfloat16)

        def remote_copy(src, dst, s, step, c, target):
            return pltpu.make_async_remote_copy(
                src_ref=src, dst_ref=dst,
                send_sem=send_sems.at[s, step, c],
                recv_sem=recv_sems.at[s, step, c],
                device_id=(target,),
                device_id_type=pl.DeviceIdType.MESH,
            )

        rs1 = [[None] * NSUB for _ in range(2)]
        for c in range(NSUB):
            for s, st in enumerate(streams):
                send_rows = (1 - st["kb"]) * half
                cc = st["c0"] + c * nq
                out_ref[pl.ds(send_rows, half), pl.ds(cc, nq)] = jnp.dot(
                    a_ref[pl.ds(send_rows, half), :].astype(jnp.bfloat16),
                    b_ref[:, cc:cc + nq].astype(jnp.bfloat16),
                    preferred_element_type=jnp.float32,
                ).astype(jnp.bfloat16)
                rdma = remote_copy(
                    out_ref.at[pl.ds(send_rows, half), pl.ds(cc, nq)],
                    r1.at[s, c], s, 0, c, st["p1"],
                )
                if s == 0 and c == 0:
                    pl.semaphore_wait(barrier_sem, 2)
                rdma.start()
                rs1[s][c] = rdma

        for st in streams:
            keep_rows = st["kb"] * half
            out_ref[pl.ds(keep_rows, half), pl.ds(st["c0"], nh)] = (
                block_dot(keep_rows, st["c0"])
            )

        rs2 = [[None] * NSUB for _ in range(2)]
        for c in range(NSUB):
            for s, st in enumerate(streams):
                keep_rows = st["kb"] * half
                cc = st["c0"] + c * nq
                rs1[s][c].wait_recv()
                out_ref[pl.ds(keep_rows, half), pl.ds(cc, nq)] = (
                    out_ref[pl.ds(keep_rows, half), pl.ds(cc, nq)]
                    + r1[s, c]
                )
                send_q = keep_rows + (1 - st["qb"]) * quart
                rdma = remote_copy(
                    out_ref.at[pl.ds(send_q, quart), pl.ds(cc, nq)],
                    r2.at[s, c], s, 1, c, st["p2"],
                )
                rdma.start()
                rs2[s][c] = rdma

        ag1 = [[None] * NSUB for _ in range(2)]
        for c in range(NSUB):
            for s, st in enumerate(streams):
                keep_q = st["kb"] * half + st["qb"] * quart
                cc = st["c0"] + c * nq
                rs2[s][c].wait_recv()
                out_ref[pl.ds(keep_q, quart), pl.ds(cc, nq)] = (
                    out_ref[pl.ds(keep_q, quart), pl.ds(cc, nq)]
                    + r2[s, c]
                )
                rdma = remote_copy(
                    out_ref.at[pl.ds(keep_q, quart), pl.ds(cc, nq)],
                    out_ref.at[pl.ds(keep_q, quart), pl.ds(cc, nq)],
                    s, 2, c, st["p2"],
                )
                rdma.start()
                ag1[s][c] = rdma

        ag2 = [[None] * NSUB for _ in range(2)]
        for c in range(NSUB):
            for s, st in enumerate(streams):
                keep_rows = st["kb"] * half
                cc = st["c0"] + c * nq
                ag1[s][c].wait_recv()
                rdma = remote_copy(
                    out_ref.at[pl.ds(keep_rows, half), pl.ds(cc, nq)],
                    out_ref.at[pl.ds(keep_rows, half), pl.ds(cc, nq)],
                    s, 3, c, st["p1"],
                )
                rdma.start()
                ag2[s][c] = rdma

        for c in range(NSUB):
            for s in range(2):
                ag2[s][c].wait_recv()

        for c in range(NSUB):
            for s in range(2):
                rs1[s][c].wait_send()
                rs2[s][c].wait_send()
                ag1[s][c].wait_send()
                ag2[s][c].wait_send()

    return pl.pallas_call(
        body,
        out_shape=jax.ShapeDtypeStruct((m, n), jnp.bfloat16),
        in_specs=[
            pl.BlockSpec(memory_space=pltpu.VMEM),
            pl.BlockSpec(memory_space=pltpu.VMEM),
        ],
        out_specs=pl.BlockSpec(memory_space=pltpu.VMEM),
        scratch_shapes=[
            pltpu.VMEM((2, NSUB, half, nq), jnp.bfloat16),
            pltpu.VMEM((2, NSUB, quart, nq), jnp.bfloat16),
            pltpu.SemaphoreType.DMA((2, 4, NSUB)),
            pltpu.SemaphoreType.DMA((2, 4, NSUB)),
        ],
        compiler_params=pltpu.CompilerParams(collective_id=0),
    )(A, B)
